# baseline (speedup 1.0000x reference)
import functools

import jax
import jax.numpy as jnp
from jax import lax
from jax.experimental import pallas as pl
from jax.experimental.pallas import tpu as pltpu
from jax.experimental.pallas import tpu_sc as plsc

_EPS = 1e-05
_MAX_VALUE = 1.0 / (1.0 + _EPS)

_NC = 2
_NS = 16
_NW = _NC * _NS

_R = 64
_DEPTH = 4


def _compute_chunk(in_v, out_v):
    lane = lax.iota(jnp.int32, 16)
    lane0 = lane == 0
    neg_inf = jnp.float32(-jnp.inf)

    def row_body(r, _):
        vs = [in_v[r, pl.ds(16 * k, 16)] for k in range(8)]
        m = jnp.where(lane0, neg_inf, vs[0])
        for k in range(1, 8):
            m = jnp.maximum(m, vs[k])
        for d in (8, 4, 2, 1):
            m = jnp.maximum(m, m.at[lane ^ d].get(mode="promise_in_bounds"))
        s = m * jnp.float32(127.0)
        scale = s / (s + jnp.float32(_EPS))
        c1 = jnp.float32(_MAX_VALUE) - m
        for k in range(8):
            o = jnp.maximum(vs[k] + c1, 0.0) * scale
            if k == 0:
                special = jnp.float32(_EPS) * (
                    jnp.maximum(vs[0] + jnp.float32(_MAX_VALUE), 0.0) + 1.0
                )
                o = jnp.where(lane0, special, o)
            out_v[r, pl.ds(16 * k, 16)] = o
        return 0

    lax.fori_loop(0, _R, row_body, 0)


def _sc_body(x_hbm, out_hbm, *scratch, rows_per_worker):
    ins = scratch[:_DEPTH]
    outs = scratch[_DEPTH:2 * _DEPTH]
    sem_is = scratch[2 * _DEPTH:3 * _DEPTH]
    sem_os = scratch[3 * _DEPTH:4 * _DEPTH]

    wid = lax.axis_index("s") * _NC + lax.axis_index("c")
    base = wid * rows_per_worker
    n_chunks = rows_per_worker // _R
    n_groups = n_chunks // _DEPTH

    def src_at(c):
        return x_hbm.at[pl.ds(base + c * _R, _R), :]

    def dst_at(c):
        return out_hbm.at[pl.ds(base + c * _R, _R), :]

    def wait_in(b):
        pltpu.make_async_copy(src_at(0), ins[b], sem_is[b]).wait()

    def wait_out(b):
        pltpu.make_async_copy(outs[b], dst_at(0), sem_os[b]).wait()

    for b in range(_DEPTH):
        pltpu.async_copy(src_at(b), ins[b], sem_is[b])

    def step(c, b, prefetch, first):
        wait_in(b)
        if not first:
            wait_out(b)
        _compute_chunk(ins[b], outs[b])
        pltpu.async_copy(outs[b], dst_at(c), sem_os[b])
        if prefetch:
            pltpu.async_copy(src_at(c + _DEPTH), ins[b], sem_is[b])

    def group_body(j, _):
        for b in range(_DEPTH):
            step(j * _DEPTH + b, b, True, False)
        return 0

    for b in range(_DEPTH):
        step(b, b, True, True)
    lax.fori_loop(1, n_groups - 1, group_body, 0)
    for b in range(_DEPTH):
        step((n_groups - 1) * _DEPTH + b, b, False, False)
    for b in range(_DEPTH):
        wait_out(b)


def kernel(X):
    N, L = X.shape
    rows_per_worker = N // _NW
    mesh = plsc.VectorSubcoreMesh(core_axis_name="c", subcore_axis_name="s")
    f = pl.kernel(
        functools.partial(_sc_body, rows_per_worker=rows_per_worker),
        mesh=mesh,
        out_type=jax.ShapeDtypeStruct((N, L), jnp.float32),
        scratch_types=(
            [pltpu.VMEM((_R, L), jnp.float32) for _ in range(2 * _DEPTH)]
            + [pltpu.SemaphoreType.DMA for _ in range(2 * _DEPTH)]
        ),
    )
    return f(X)
